# Initial kernel scaffold; baseline (speedup 1.0000x reference)
import jax
import jax.numpy as jnp
from jax.experimental import pallas as pl


def kernel(ctc_emissions, ctc_transition, ctc_bos, ctc_eos, targets):
    raise NotImplementedError("write your pallas kernel here")


# speedup vs baseline: 488.5500x; 488.5500x over previous
import jax
import jax.numpy as jnp
from jax.experimental import pallas as pl
from jax.experimental.pallas import tpu as pltpu

NEG = -1e30


def _lae(a, b):
    mx = jnp.maximum(a, b)
    return mx + jnp.log1p(jnp.exp(-jnp.abs(a - b)))


def _fwd_kernel(em_ref, transT_ref, bos_ref, eos_ref, tgt_ref, out_ref,
                m_ref, ty_ref):
    T = em_ref.shape[0]
    B, L = tgt_ref.shape
    tgt = tgt_ref[...]
    for k in range(4):
        m_ref[k] = (tgt == k).astype(jnp.float32)
    transT = transT_ref[...]
    lane = jax.lax.broadcasted_iota(jnp.int32, (B, L), 1)

    tgt_prev = pltpu.roll(tgt, 1, axis=1)
    ty = jnp.zeros((B, L), jnp.float32)
    for i in range(4):
        mi = (tgt_prev == i).astype(jnp.float32)
        for j in range(4):
            ty = ty + (mi * m_ref[j]) * transT[j:j + 1, i:i + 1]
    ty = jnp.where(lane == 0, 0.0, ty)
    ty_ref[...] = ty

    em0 = em_ref[0]
    bos_row = bos_ref[...]
    eos_row = eos_ref[...]
    a_den = bos_row + em0[:, :4]
    first = jnp.zeros((B, 1), jnp.float32)
    for k in range(4):
        first = first + m_ref[k][:, 0:1] * a_den[:, k:k + 1]
    a_num = jnp.where(lane == 0, first, NEG)

    def body(t, carry):
        a_num, a_den = carry
        em_t = em_ref[t]
        ent = em_t[:, :4]
        ext = em_t[:, 4:]
        e_en = m_ref[0] * ent[:, 0:1] + m_ref[1] * ent[:, 1:2] \
            + m_ref[2] * ent[:, 2:3] + m_ref[3] * ent[:, 3:4]
        e_ex = m_ref[0] * ext[:, 0:1] + m_ref[1] * ext[:, 1:2] \
            + m_ref[2] * ext[:, 2:3] + m_ref[3] * ext[:, 3:4]
        stay = a_num + e_ex
        sh = jnp.where(lane == 0, NEG, pltpu.roll(a_num, 1, axis=1))
        move = sh + ty_ref[...] + e_en
        a_num = _lae(stay, move)
        cols = []
        for j in range(4):
            s = a_den + transT[j:j + 1, :]
            mx = jnp.max(s, axis=1, keepdims=True)
            cols.append(mx + jnp.log(jnp.sum(jnp.exp(s - mx), axis=1,
                                             keepdims=True)))
        mv = jnp.concatenate(cols, axis=1) + ent
        a_den = _lae(a_den + ext, mv)
        return a_num, a_den

    a_num, a_den = jax.lax.fori_loop(1, T, body, (a_num, a_den))

    s = a_den + eos_row
    mx = jnp.max(s, axis=1, keepdims=True)
    logz_den = mx + jnp.log(jnp.sum(jnp.exp(s - mx), axis=1, keepdims=True))
    eos_sel = jnp.zeros((B, 1), jnp.float32)
    for k in range(4):
        eos_sel = eos_sel + m_ref[k][:, L - 1:L] * eos_row[:, k:k + 1]
    logz_num = a_num[:, L - 1:L] + eos_sel
    out_ref[...] = (jnp.sum(logz_den - logz_num) / B).reshape(1, 1)


def kernel(ctc_emissions, ctc_transition, ctc_bos, ctc_eos, targets):
    em = ctc_emissions.astype(jnp.float32)
    B, T, _ = em.shape
    L = targets.shape[1]
    em_t = jnp.transpose(em, (1, 0, 2))
    transT = ctc_transition.astype(jnp.float32).T
    bos = ctc_bos.astype(jnp.float32).reshape(1, 4)
    eos = ctc_eos.astype(jnp.float32).reshape(1, 4)
    tgt = targets.astype(jnp.int32)
    out = pl.pallas_call(
        _fwd_kernel,
        out_shape=jax.ShapeDtypeStruct((1, 1), jnp.float32),
        scratch_shapes=[
            pltpu.VMEM((4, B, L), jnp.float32),
            pltpu.VMEM((B, L), jnp.float32),
        ],
    )(em_t, transT, bos, eos, tgt)
    return out[0, 0]
